# baseline (speedup 1.0000x reference)
import dataclasses
import functools

import jax
import jax.numpy as jnp
from jax import lax
from jax.experimental import pallas as pl
from jax.experimental.pallas import tpu as pltpu
from jax.experimental.pallas import tpu_sc as plsc

_N = 10000
_E = 320000
_D_IN = 128
_HID = 128
_C = 64

_NC = 2
_NS = 16
_NW = _NC * _NS
_L = 16

_N_PAD = 10240
_CHUNK = 128
_CH = 80
_E_PAD = _NW * _CH * _CHUNK
_RB = 1024

_H = _N_PAD // 2
_TR = _H
_H_ACC = 5248
_SRZ = _H_ACC // _NS
_SRO = _H // _NS
_CHP = 84
_CAPP = _CHP * _CHUNK
_NBUF = 4

_mesh = plsc.VectorSubcoreMesh(core_axis_name="c", subcore_axis_name="s")

_sc_params = pltpu.CompilerParams()
if "needs_layout_passes" in pltpu.CompilerParams.__dataclass_fields__:
    _sc_params = dataclasses.replace(_sc_params, needs_layout_passes=False)
_sc_linear_params = _sc_params
if "use_tc_tiling_on_sc" in pltpu.CompilerParams.__dataclass_fields__:
    _sc_linear_params = dataclasses.replace(
        _sc_linear_params, use_tc_tiling_on_sc=False)


def _sc_partition_degree(src_hbm, dst_hbm):
    i32 = jnp.int32

    @functools.partial(
        pl.kernel,
        out_type=[
            jax.ShapeDtypeStruct((_NW, _N_PAD), jnp.float32),
            jax.ShapeDtypeStruct((_NW, _CAPP), i32),
            jax.ShapeDtypeStruct((_NW, _CAPP), i32),
            jax.ShapeDtypeStruct((_NW, _CAPP), i32),
            jax.ShapeDtypeStruct((_NW, _CAPP), i32),
        ],
        mesh=_mesh,
        compiler_params=_sc_params,
        scratch_types=[
            pltpu.VMEM((_CH, _CHUNK), i32),
            pltpu.VMEM((_CH, _CHUNK), i32),
            pltpu.VMEM((_CAPP,), i32),
            pltpu.VMEM((_CAPP,), i32),
            pltpu.VMEM((_CAPP,), i32),
            pltpu.VMEM((_CAPP,), i32),
            pltpu.VMEM((_N_PAD,), jnp.float32),
        ],
    )
    def k(src_idx, dst_idx, deg_hbm, s0_hbm, d0_hbm, s1_hbm, d1_hbm,
          src_v, dst_v, ps0, pd0, ps1, pd1, hist):
        c = lax.axis_index("c")
        s = lax.axis_index("s")
        w = c * _NS + s

        trash_src = jnp.full((_L,), _N, i32)
        trash_dst = jnp.full((_L,), _TR, i32)

        @pl.loop(0, _CAPP, step=_L)
        def _(i):
            ps0[pl.ds(i, _L)] = trash_src
            pd0[pl.ds(i, _L)] = trash_dst
            ps1[pl.ds(i, _L)] = trash_src
            pd1[pl.ds(i, _L)] = trash_dst

        @pl.loop(0, _N_PAD, step=_L)
        def _(i):
            hist[pl.ds(i, _L)] = jnp.zeros((_L,), jnp.float32)

        pltpu.sync_copy(src_idx.at[w], src_v)
        pltpu.sync_copy(dst_idx.at[w], dst_v)

        ones = jnp.full((_L,), 1.0, jnp.float32)
        hvec = jnp.full((_L,), _H, i32)

        def vec_body(i, curs):
            c0, c1 = curs
            j = i // (_CHUNK // _L)
            kk = (i % (_CHUNK // _L)) * _L
            vs = src_v[j, pl.ds(kk, _L)]
            vd = dst_v[j, pl.ds(kk, _L)]
            plsc.addupdate_scatter(hist, [vd], ones)
            m0 = vd < hvec
            n0 = jnp.max(plsc.all_reduce_population_count(m0))
            plsc.store_compressed(ps0.at[pl.ds(c0, _L)], vs, mask=m0)
            plsc.store_compressed(pd0.at[pl.ds(c0, _L)], vd, mask=m0)
            m1 = jnp.logical_not(m0)
            plsc.store_compressed(ps1.at[pl.ds(c1, _L)], vs, mask=m1)
            plsc.store_compressed(pd1.at[pl.ds(c1, _L)], vd - hvec, mask=m1)
            return (c0 + n0, c1 + (_L - n0))

        lax.fori_loop(0, _CH * (_CHUNK // _L), vec_body,
                      (jnp.int32(0), jnp.int32(0)))

        pltpu.sync_copy(hist, deg_hbm.at[w])
        pltpu.sync_copy(ps0, s0_hbm.at[w])
        pltpu.sync_copy(pd0, d0_hbm.at[w])
        pltpu.sync_copy(ps1, s1_hbm.at[w])
        pltpu.sync_copy(pd1, d1_hbm.at[w])

    return k(src_hbm, dst_hbm)


def _sc_aggregate(g_hbm, srcl_hbm, dstl_hbm, d, linear=False):
    @functools.partial(
        pl.kernel,
        out_type=jax.ShapeDtypeStruct((_N_PAD, d), jnp.float32),
        mesh=_mesh,
        compiler_params=_sc_linear_params if linear else _sc_params,
        scratch_types=[
            pltpu.VMEM((_CHP, _CHUNK), jnp.int32),
            pltpu.VMEM((_CHP, _CHUNK), jnp.int32),
            pltpu.VMEM((_CHUNK, d), jnp.float32),
            pltpu.VMEM((_CHUNK, d), jnp.float32),
            pltpu.VMEM((_CHUNK, d), jnp.float32),
            pltpu.VMEM((_CHUNK, d), jnp.float32),
            pltpu.VMEM_SHARED((_H_ACC, d), jnp.float32),
            pltpu.SemaphoreType.DMA,
            pltpu.SemaphoreType.DMA,
            pltpu.SemaphoreType.DMA,
            pltpu.SemaphoreType.DMA,
        ],
    )
    def k(g, src_idx, dst_idx, out_hbm, src_v, dst_v, r0, r1, r2, r3, acc,
          gsem, gsem2, gsem3, gsem4):
        rows = (r0, r1, r2, r3)
        gsems = (gsem, gsem2, gsem3, gsem4)
        c = lax.axis_index("c")
        s = lax.axis_index("s")

        @pl.loop(0, _CHUNK)
        def _(i):
            @pl.loop(0, d, step=_L)
            def _(kk):
                r0[i, pl.ds(kk, _L)] = jnp.zeros((_L,), jnp.float32)

        @pl.loop(0, _SRZ // _CHUNK)
        def _(t):
            pltpu.sync_copy(r0, acc.at[pl.ds(s * _SRZ + t * _CHUNK, _CHUNK)])
        pltpu.sync_copy(
            r0.at[pl.ds(0, _SRZ % _CHUNK)],
            acc.at[pl.ds(s * _SRZ + (_SRZ // _CHUNK) * _CHUNK,
                         _SRZ % _CHUNK)])
        plsc.subcore_barrier()

        for sub in range(2):
            lidx = c * _NW + 2 * s + sub
            pltpu.sync_copy(src_idx.at[lidx], src_v)
            pltpu.sync_copy(dst_idx.at[lidx], dst_v)

            def count_body(j, nv):
                lead = dst_v[j, pl.ds(0, _L)]
                return nv + jnp.where(jnp.min(lead) < _TR,
                                      jnp.int32(1), jnp.int32(0))

            nv = lax.fori_loop(0, _CHP, count_body, jnp.int32(0))
            ng = (nv + _NBUF - 1) // _NBUF

            def grp_body(t, carry):
                base = t * _NBUF
                for b in range(_NBUF):
                    pltpu.async_copy(g.at[src_v.at[base + b]], rows[b],
                                     gsems[b])
                for b in range(_NBUF):
                    pltpu.make_async_copy(g.at[src_v.at[0]], rows[b],
                                          gsems[b]).wait()
                    pltpu.sync_copy(rows[b],
                                    acc.at[dst_v.at[base + b]], add=True)
                return carry

            lax.fori_loop(0, ng, grp_body, jnp.int32(0))

        plsc.subcore_barrier()
        pltpu.sync_copy(acc.at[pl.ds(s * _SRO, _SRO)],
                        out_hbm.at[pl.ds(c * _H + s * _SRO, _SRO)])

    return k(g_hbm, srcl_hbm, dstl_hbm)


def _mm_body(x_ref, w_ref, o_ref):
    o_ref[...] = jnp.dot(x_ref[...], w_ref[...],
                         preferred_element_type=jnp.float32,
                         precision=lax.Precision.HIGHEST)


def _tc_matmul(x_pad, W):
    d_in, d_out = W.shape
    return pl.pallas_call(
        _mm_body,
        grid=(_N_PAD // _RB,),
        in_specs=[pl.BlockSpec((_RB, d_in), lambda i: (i, 0)),
                  pl.BlockSpec((d_in, d_out), lambda i: (0, 0))],
        out_specs=pl.BlockSpec((_RB, d_out), lambda i: (i, 0)),
        out_shape=jax.ShapeDtypeStruct((_N_PAD, d_out), jnp.float32),
    )(x_pad, W)


def _dis_scale_body(dt_ref, h_ref, dis_ref, g_ref):
    dsum = jnp.sum(dt_ref[...], axis=1, keepdims=True) + 1.0
    dis = lax.rsqrt(dsum)
    dis_ref[...] = dis
    g_ref[...] = h_ref[...] * dis


def _tc_dis_scale(deg_t, h1):
    return pl.pallas_call(
        _dis_scale_body,
        grid=(_N_PAD // _RB,),
        in_specs=[pl.BlockSpec((_RB, _NW), lambda i: (i, 0)),
                  pl.BlockSpec((_RB, _HID), lambda i: (i, 0))],
        out_specs=[pl.BlockSpec((_RB, 1), lambda i: (i, 0)),
                   pl.BlockSpec((_RB, _HID), lambda i: (i, 0))],
        out_shape=[jax.ShapeDtypeStruct((_N_PAD, 1), jnp.float32),
                   jax.ShapeDtypeStruct((_N_PAD, _HID), jnp.float32)],
    )(deg_t, h1)


def _layer1_body(p_ref, g1_ref, dis_ref, b1_ref, w2_ref, g2_ref):
    ssum = p_ref[...] + g1_ref[...]
    z = jnp.maximum(ssum * dis_ref[...] + b1_ref[...], 0.0)
    h2 = jnp.dot(z, w2_ref[...], preferred_element_type=jnp.float32,
                 precision=lax.Precision.HIGHEST)
    g2_ref[...] = h2 * dis_ref[...]


def _tc_layer1_combine(p, g1, dis, b1, W2):
    return pl.pallas_call(
        _layer1_body,
        grid=(_N_PAD // _RB,),
        in_specs=[pl.BlockSpec((_RB, _HID), lambda i: (i, 0)),
                  pl.BlockSpec((_RB, _HID), lambda i: (i, 0)),
                  pl.BlockSpec((_RB, 1), lambda i: (i, 0)),
                  pl.BlockSpec((1, _HID), lambda i: (0, 0)),
                  pl.BlockSpec((_HID, _C), lambda i: (0, 0))],
        out_specs=pl.BlockSpec((_RB, _C), lambda i: (i, 0)),
        out_shape=jax.ShapeDtypeStruct((_N_PAD, _C), jnp.float32),
    )(p, g1, dis, b1, W2)


def _layer2_body(q_ref, g2_ref, dis_ref, b2_ref, o_ref):
    ssum = q_ref[...] + g2_ref[...]
    o_ref[...] = ssum * dis_ref[...] + b2_ref[...]


def _tc_layer2_combine(q, g2, dis, b2):
    return pl.pallas_call(
        _layer2_body,
        grid=(_N_PAD // _RB,),
        in_specs=[pl.BlockSpec((_RB, _C), lambda i: (i, 0)),
                  pl.BlockSpec((_RB, _C), lambda i: (i, 0)),
                  pl.BlockSpec((_RB, 1), lambda i: (i, 0)),
                  pl.BlockSpec((1, _C), lambda i: (0, 0))],
        out_specs=pl.BlockSpec((_RB, _C), lambda i: (i, 0)),
        out_shape=jax.ShapeDtypeStruct((_N_PAD, _C), jnp.float32),
    )(q, g2, dis, b2)


def kernel(x, edge_index, W1, b1, W2, b2):
    src = edge_index[0]
    dst = edge_index[1]
    npad = _E_PAD - _E
    pad_idx = jnp.full((npad,), _N, jnp.int32)
    src_p = jnp.concatenate([src, pad_idx]).reshape(_NW, _CH, _CHUNK)
    dst_p = jnp.concatenate([dst, pad_idx]).reshape(_NW, _CH, _CHUNK)
    x_pad = jnp.zeros((_N_PAD, _D_IN), jnp.float32).at[:_N].set(x)

    deg, s0, d0, s1, d1 = _sc_partition_degree(src_p, dst_p)
    deg_t = deg.T
    srcl = jnp.concatenate([s0, s1]).reshape(2 * _NW, _CHP, _CHUNK)
    dstl = jnp.concatenate([d0, d1]).reshape(2 * _NW, _CHP, _CHUNK)

    h1 = _tc_matmul(x_pad, W1)
    dis, g1 = _tc_dis_scale(deg_t, h1)

    p = _sc_aggregate(g1, srcl, dstl, _HID)
    g2 = _tc_layer1_combine(p, g1, dis, b1.reshape(1, _HID), W2)

    q = _sc_aggregate(g2, srcl, dstl, _C, linear=True)
    out = _tc_layer2_combine(q, g2, dis, b2.reshape(1, _C))
    return out[:_N]


# speedup vs baseline: 1.0006x; 1.0006x over previous
import dataclasses
import functools

import jax
import jax.numpy as jnp
from jax import lax
from jax.experimental import pallas as pl
from jax.experimental.pallas import tpu as pltpu
from jax.experimental.pallas import tpu_sc as plsc

_N = 10000
_E = 320000
_D_IN = 128
_HID = 128
_C = 64

_NC = 2
_NS = 16
_NW = _NC * _NS
_L = 16

_N_PAD = 10240
_CHUNK = 128
_CH = 80
_E_PAD = _NW * _CH * _CHUNK
_RB = 1024

_H = _N_PAD // 2
_TR = _H
_H_ACC = 5248
_SRZ = _H_ACC // _NS
_SRO = _H // _NS
_CHP = 84
_CAPP = _CHP * _CHUNK
_NBUF = 4

_mesh = plsc.VectorSubcoreMesh(core_axis_name="c", subcore_axis_name="s")

_sc_params = pltpu.CompilerParams()
if "needs_layout_passes" in pltpu.CompilerParams.__dataclass_fields__:
    _sc_params = dataclasses.replace(_sc_params, needs_layout_passes=False)
_sc_linear_params = _sc_params
if "use_tc_tiling_on_sc" in pltpu.CompilerParams.__dataclass_fields__:
    _sc_linear_params = dataclasses.replace(
        _sc_linear_params, use_tc_tiling_on_sc=False)


def _sc_partition_degree(src_hbm, dst_hbm):
    i32 = jnp.int32

    @functools.partial(
        pl.kernel,
        out_type=[
            jax.ShapeDtypeStruct((_NW, _N_PAD), jnp.float32),
            jax.ShapeDtypeStruct((_NW, _CAPP), i32),
            jax.ShapeDtypeStruct((_NW, _CAPP), i32),
            jax.ShapeDtypeStruct((_NW, _CAPP), i32),
            jax.ShapeDtypeStruct((_NW, _CAPP), i32),
        ],
        mesh=_mesh,
        compiler_params=_sc_params,
        scratch_types=[
            pltpu.VMEM((_CH, _CHUNK), i32),
            pltpu.VMEM((_CH, _CHUNK), i32),
            pltpu.VMEM((_CAPP,), i32),
            pltpu.VMEM((_CAPP,), i32),
            pltpu.VMEM((_CAPP,), i32),
            pltpu.VMEM((_CAPP,), i32),
            pltpu.VMEM((_N_PAD,), jnp.float32),
        ],
    )
    def k(src_idx, dst_idx, deg_hbm, s0_hbm, d0_hbm, s1_hbm, d1_hbm,
          src_v, dst_v, ps0, pd0, ps1, pd1, hist):
        c = lax.axis_index("c")
        s = lax.axis_index("s")
        w = c * _NS + s

        trash_src = jnp.full((_L,), _N, i32)
        trash_dst = jnp.full((_L,), _TR, i32)

        @pl.loop(0, _CAPP, step=_L)
        def _(i):
            ps0[pl.ds(i, _L)] = trash_src
            pd0[pl.ds(i, _L)] = trash_dst
            ps1[pl.ds(i, _L)] = trash_src
            pd1[pl.ds(i, _L)] = trash_dst

        @pl.loop(0, _N_PAD, step=_L)
        def _(i):
            hist[pl.ds(i, _L)] = jnp.zeros((_L,), jnp.float32)

        pltpu.sync_copy(src_idx.at[w], src_v)
        pltpu.sync_copy(dst_idx.at[w], dst_v)

        ones = jnp.full((_L,), 1.0, jnp.float32)
        hvec = jnp.full((_L,), _H, i32)

        def vec_body(i, curs):
            c0, c1 = curs
            j = i // (_CHUNK // _L)
            kk = (i % (_CHUNK // _L)) * _L
            vs = src_v[j, pl.ds(kk, _L)]
            vd = dst_v[j, pl.ds(kk, _L)]
            plsc.addupdate_scatter(hist, [vd], ones)
            m0 = vd < hvec
            n0 = jnp.max(plsc.all_reduce_population_count(m0))
            plsc.store_compressed(ps0.at[pl.ds(c0, _L)], vs, mask=m0)
            plsc.store_compressed(pd0.at[pl.ds(c0, _L)], vd, mask=m0)
            m1 = jnp.logical_not(m0)
            plsc.store_compressed(ps1.at[pl.ds(c1, _L)], vs, mask=m1)
            plsc.store_compressed(pd1.at[pl.ds(c1, _L)], vd - hvec, mask=m1)
            return (c0 + n0, c1 + (_L - n0))

        lax.fori_loop(0, _CH * (_CHUNK // _L), vec_body,
                      (jnp.int32(0), jnp.int32(0)))

        pltpu.sync_copy(hist, deg_hbm.at[w])
        pltpu.sync_copy(ps0, s0_hbm.at[w])
        pltpu.sync_copy(pd0, d0_hbm.at[w])
        pltpu.sync_copy(ps1, s1_hbm.at[w])
        pltpu.sync_copy(pd1, d1_hbm.at[w])

    return k(src_hbm, dst_hbm)


def _sc_aggregate(g_hbm, srcl_hbm, dstl_hbm, d, linear=False):
    @functools.partial(
        pl.kernel,
        out_type=jax.ShapeDtypeStruct((_N_PAD, d), jnp.float32),
        mesh=_mesh,
        compiler_params=_sc_linear_params if linear else _sc_params,
        scratch_types=[
            pltpu.VMEM((_CHP, _CHUNK), jnp.int32),
            pltpu.VMEM((_CHP, _CHUNK), jnp.int32),
            pltpu.VMEM((_CHUNK, d), jnp.float32),
            pltpu.VMEM((_CHUNK, d), jnp.float32),
            pltpu.VMEM((_CHUNK, d), jnp.float32),
            pltpu.VMEM((_CHUNK, d), jnp.float32),
            pltpu.VMEM_SHARED((_H_ACC, d), jnp.float32),
            pltpu.SemaphoreType.DMA,
            pltpu.SemaphoreType.DMA,
            pltpu.SemaphoreType.DMA,
            pltpu.SemaphoreType.DMA,
        ],
    )
    def k(g, src_idx, dst_idx, out_hbm, src_v, dst_v, r0, r1, r2, r3, acc,
          gsem, gsem2, gsem3, gsem4):
        rows = (r0, r1, r2, r3)
        gsems = (gsem, gsem2, gsem3, gsem4)
        c = lax.axis_index("c")
        s = lax.axis_index("s")

        @pl.loop(0, _CHUNK)
        def _(i):
            @pl.loop(0, d, step=_L)
            def _(kk):
                r0[i, pl.ds(kk, _L)] = jnp.zeros((_L,), jnp.float32)

        @pl.loop(0, _SRZ // _CHUNK)
        def _(t):
            pltpu.sync_copy(r0, acc.at[pl.ds(s * _SRZ + t * _CHUNK, _CHUNK)])
        pltpu.sync_copy(
            r0.at[pl.ds(0, _SRZ % _CHUNK)],
            acc.at[pl.ds(s * _SRZ + (_SRZ // _CHUNK) * _CHUNK,
                         _SRZ % _CHUNK)])
        plsc.subcore_barrier()

        for sub in range(2):
            lidx = c * _NW + 2 * s + sub
            pltpu.sync_copy(src_idx.at[lidx], src_v)
            pltpu.sync_copy(dst_idx.at[lidx], dst_v)

            def count_body(j, nv):
                lead = dst_v[j, pl.ds(0, _L)]
                return nv + jnp.where(jnp.min(lead) < _TR,
                                      jnp.int32(1), jnp.int32(0))

            nv = lax.fori_loop(0, _CHP, count_body, jnp.int32(0))

            @pl.loop(0, _CHP // _NBUF)
            def _(t):
                @pl.when(t * _NBUF < nv)
                def _():
                    base = t * _NBUF
                    cps = [pltpu.async_copy(g.at[src_v.at[base + b]],
                                            rows[b], gsems[b])
                           for b in range(_NBUF)]
                    for b in range(_NBUF):
                        cps[b].wait()
                        pltpu.sync_copy(rows[b],
                                        acc.at[dst_v.at[base + b]], add=True)

        plsc.subcore_barrier()
        pltpu.sync_copy(acc.at[pl.ds(s * _SRO, _SRO)],
                        out_hbm.at[pl.ds(c * _H + s * _SRO, _SRO)])

    return k(g_hbm, srcl_hbm, dstl_hbm)


def _mm_body(x_ref, w_ref, o_ref):
    o_ref[...] = jnp.dot(x_ref[...], w_ref[...],
                         preferred_element_type=jnp.float32,
                         precision=lax.Precision.HIGHEST)


def _tc_matmul(x_pad, W):
    d_in, d_out = W.shape
    return pl.pallas_call(
        _mm_body,
        grid=(_N_PAD // _RB,),
        in_specs=[pl.BlockSpec((_RB, d_in), lambda i: (i, 0)),
                  pl.BlockSpec((d_in, d_out), lambda i: (0, 0))],
        out_specs=pl.BlockSpec((_RB, d_out), lambda i: (i, 0)),
        out_shape=jax.ShapeDtypeStruct((_N_PAD, d_out), jnp.float32),
    )(x_pad, W)


def _dis_scale_body(dt_ref, h_ref, dis_ref, g_ref):
    dsum = jnp.sum(dt_ref[...], axis=1, keepdims=True) + 1.0
    dis = lax.rsqrt(dsum)
    dis_ref[...] = dis
    g_ref[...] = h_ref[...] * dis


def _tc_dis_scale(deg_t, h1):
    return pl.pallas_call(
        _dis_scale_body,
        grid=(_N_PAD // _RB,),
        in_specs=[pl.BlockSpec((_RB, _NW), lambda i: (i, 0)),
                  pl.BlockSpec((_RB, _HID), lambda i: (i, 0))],
        out_specs=[pl.BlockSpec((_RB, 1), lambda i: (i, 0)),
                   pl.BlockSpec((_RB, _HID), lambda i: (i, 0))],
        out_shape=[jax.ShapeDtypeStruct((_N_PAD, 1), jnp.float32),
                   jax.ShapeDtypeStruct((_N_PAD, _HID), jnp.float32)],
    )(deg_t, h1)


def _layer1_body(p_ref, g1_ref, dis_ref, b1_ref, w2_ref, g2_ref):
    ssum = p_ref[...] + g1_ref[...]
    z = jnp.maximum(ssum * dis_ref[...] + b1_ref[...], 0.0)
    h2 = jnp.dot(z, w2_ref[...], preferred_element_type=jnp.float32,
                 precision=lax.Precision.HIGHEST)
    g2_ref[...] = h2 * dis_ref[...]


def _tc_layer1_combine(p, g1, dis, b1, W2):
    return pl.pallas_call(
        _layer1_body,
        grid=(_N_PAD // _RB,),
        in_specs=[pl.BlockSpec((_RB, _HID), lambda i: (i, 0)),
                  pl.BlockSpec((_RB, _HID), lambda i: (i, 0)),
                  pl.BlockSpec((_RB, 1), lambda i: (i, 0)),
                  pl.BlockSpec((1, _HID), lambda i: (0, 0)),
                  pl.BlockSpec((_HID, _C), lambda i: (0, 0))],
        out_specs=pl.BlockSpec((_RB, _C), lambda i: (i, 0)),
        out_shape=jax.ShapeDtypeStruct((_N_PAD, _C), jnp.float32),
    )(p, g1, dis, b1, W2)


def _layer2_body(q_ref, g2_ref, dis_ref, b2_ref, o_ref):
    ssum = q_ref[...] + g2_ref[...]
    o_ref[...] = ssum * dis_ref[...] + b2_ref[...]


def _tc_layer2_combine(q, g2, dis, b2):
    return pl.pallas_call(
        _layer2_body,
        grid=(_N_PAD // _RB,),
        in_specs=[pl.BlockSpec((_RB, _C), lambda i: (i, 0)),
                  pl.BlockSpec((_RB, _C), lambda i: (i, 0)),
                  pl.BlockSpec((_RB, 1), lambda i: (i, 0)),
                  pl.BlockSpec((1, _C), lambda i: (0, 0))],
        out_specs=pl.BlockSpec((_RB, _C), lambda i: (i, 0)),
        out_shape=jax.ShapeDtypeStruct((_N_PAD, _C), jnp.float32),
    )(q, g2, dis, b2)


def kernel(x, edge_index, W1, b1, W2, b2):
    src = edge_index[0]
    dst = edge_index[1]
    npad = _E_PAD - _E
    pad_idx = jnp.full((npad,), _N, jnp.int32)
    src_p = jnp.concatenate([src, pad_idx]).reshape(_NW, _CH, _CHUNK)
    dst_p = jnp.concatenate([dst, pad_idx]).reshape(_NW, _CH, _CHUNK)
    x_pad = jnp.zeros((_N_PAD, _D_IN), jnp.float32).at[:_N].set(x)

    deg, s0, d0, s1, d1 = _sc_partition_degree(src_p, dst_p)
    deg_t = deg.T
    srcl = jnp.concatenate([s0, s1]).reshape(2 * _NW, _CHP, _CHUNK)
    dstl = jnp.concatenate([d0, d1]).reshape(2 * _NW, _CHP, _CHUNK)

    h1 = _tc_matmul(x_pad, W1)
    dis, g1 = _tc_dis_scale(deg_t, h1)

    p = _sc_aggregate(g1, srcl, dstl, _HID)
    g2 = _tc_layer1_combine(p, g1, dis, b1.reshape(1, _HID), W2)

    q = _sc_aggregate(g2, srcl, dstl, _C, linear=True)
    out = _tc_layer2_combine(q, g2, dis, b2.reshape(1, _C))
    return out[:_N]


# speedup vs baseline: 1.4785x; 1.4776x over previous
import dataclasses
import functools

import jax
import jax.numpy as jnp
from jax import lax
from jax.experimental import pallas as pl
from jax.experimental.pallas import tpu as pltpu
from jax.experimental.pallas import tpu_sc as plsc

_N = 10000
_E = 320000
_D_IN = 128
_HID = 128
_C = 64

_NC = 2
_NS = 16
_NW = _NC * _NS
_L = 16

_N_PAD = 10240
_CHUNK = 128
_CH = 80
_E_PAD = _NW * _CH * _CHUNK
_RB = 1024

_H = _N_PAD // 2
_TR = _H
_H_ACC = 5248
_SRZ = _H_ACC // _NS
_SRO = _H // _NS
_CHP = 84
_CAPP = _CHP * _CHUNK
_NBUF = 4

_mesh = plsc.VectorSubcoreMesh(core_axis_name="c", subcore_axis_name="s")

_sc_params = pltpu.CompilerParams()
if "needs_layout_passes" in pltpu.CompilerParams.__dataclass_fields__:
    _sc_params = dataclasses.replace(_sc_params, needs_layout_passes=False)
_sc_linear_params = _sc_params
if "use_tc_tiling_on_sc" in pltpu.CompilerParams.__dataclass_fields__:
    _sc_linear_params = dataclasses.replace(
        _sc_linear_params, use_tc_tiling_on_sc=False)


def _sc_partition_degree(src_hbm, dst_hbm):
    i32 = jnp.int32

    @functools.partial(
        pl.kernel,
        out_type=[
            jax.ShapeDtypeStruct((_NW, _N_PAD), jnp.float32),
            jax.ShapeDtypeStruct((_NW, _CAPP), i32),
            jax.ShapeDtypeStruct((_NW, _CAPP), i32),
            jax.ShapeDtypeStruct((_NW, _CAPP), i32),
            jax.ShapeDtypeStruct((_NW, _CAPP), i32),
        ],
        mesh=_mesh,
        compiler_params=_sc_params,
        scratch_types=[
            pltpu.VMEM((_CH, _CHUNK), i32),
            pltpu.VMEM((_CH, _CHUNK), i32),
            pltpu.VMEM((_CAPP,), i32),
            pltpu.VMEM((_CAPP,), i32),
            pltpu.VMEM((_CAPP,), i32),
            pltpu.VMEM((_CAPP,), i32),
            pltpu.VMEM((_N_PAD,), jnp.float32),
        ],
    )
    def k(src_idx, dst_idx, deg_hbm, s0_hbm, d0_hbm, s1_hbm, d1_hbm,
          src_v, dst_v, ps0, pd0, ps1, pd1, hist):
        c = lax.axis_index("c")
        s = lax.axis_index("s")
        w = c * _NS + s

        trash_src = jnp.full((_L,), _N, i32)
        trash_dst = jnp.full((_L,), _TR, i32)

        @pl.loop(0, _CAPP, step=_L)
        def _(i):
            ps0[pl.ds(i, _L)] = trash_src
            pd0[pl.ds(i, _L)] = trash_dst
            ps1[pl.ds(i, _L)] = trash_src
            pd1[pl.ds(i, _L)] = trash_dst

        @pl.loop(0, _N_PAD, step=_L)
        def _(i):
            hist[pl.ds(i, _L)] = jnp.zeros((_L,), jnp.float32)

        pltpu.sync_copy(src_idx.at[w], src_v)
        pltpu.sync_copy(dst_idx.at[w], dst_v)

        ones = jnp.full((_L,), 1.0, jnp.float32)
        hvec = jnp.full((_L,), _H, i32)

        def vec_body(i, curs):
            c0, c1 = curs
            j = i // (_CHUNK // _L)
            kk = (i % (_CHUNK // _L)) * _L
            vs = src_v[j, pl.ds(kk, _L)]
            vd = dst_v[j, pl.ds(kk, _L)]
            plsc.addupdate_scatter(hist, [vd], ones)
            m0 = vd < hvec
            n0 = jnp.max(plsc.all_reduce_population_count(m0))
            plsc.store_compressed(ps0.at[pl.ds(c0, _L)], vs, mask=m0)
            plsc.store_compressed(pd0.at[pl.ds(c0, _L)], vd, mask=m0)
            m1 = jnp.logical_not(m0)
            plsc.store_compressed(ps1.at[pl.ds(c1, _L)], vs, mask=m1)
            plsc.store_compressed(pd1.at[pl.ds(c1, _L)], vd - hvec, mask=m1)
            return (c0 + n0, c1 + (_L - n0))

        lax.fori_loop(0, _CH * (_CHUNK // _L), vec_body,
                      (jnp.int32(0), jnp.int32(0)))

        pltpu.sync_copy(hist, deg_hbm.at[w])
        pltpu.sync_copy(ps0, s0_hbm.at[w])
        pltpu.sync_copy(pd0, d0_hbm.at[w])
        pltpu.sync_copy(ps1, s1_hbm.at[w])
        pltpu.sync_copy(pd1, d1_hbm.at[w])

    return k(src_hbm, dst_hbm)


def _sc_aggregate(g_hbm, srcl_hbm, dstl_hbm, d, linear=False, pipelined=True):
    @functools.partial(
        pl.kernel,
        out_type=jax.ShapeDtypeStruct((_N_PAD, d), jnp.float32),
        mesh=_mesh,
        compiler_params=_sc_linear_params if linear else _sc_params,
        scratch_types=[
            pltpu.VMEM((_CHP, _CHUNK), jnp.int32),
            pltpu.VMEM((_CHP, _CHUNK), jnp.int32),
            pltpu.VMEM((_CHUNK, d), jnp.float32),
            pltpu.VMEM((_CHUNK, d), jnp.float32),
            pltpu.VMEM((_CHUNK, d), jnp.float32),
            pltpu.VMEM((_CHUNK, d), jnp.float32),
            pltpu.VMEM_SHARED((_H_ACC, d), jnp.float32),
            pltpu.SemaphoreType.DMA,
            pltpu.SemaphoreType.DMA,
            pltpu.SemaphoreType.DMA,
            pltpu.SemaphoreType.DMA,
        ],
    )
    def k(g, src_idx, dst_idx, out_hbm, src_v, dst_v, r0, r1, r2, r3, acc,
          gsem, gsem2, gsem3, gsem4):
        rows = (r0, r1, r2, r3)
        gsems = (gsem, gsem2, gsem3, gsem4)
        c = lax.axis_index("c")
        s = lax.axis_index("s")

        @pl.loop(0, _CHUNK)
        def _(i):
            @pl.loop(0, d, step=_L)
            def _(kk):
                r0[i, pl.ds(kk, _L)] = jnp.zeros((_L,), jnp.float32)

        @pl.loop(0, _SRZ // _CHUNK)
        def _(t):
            pltpu.sync_copy(r0, acc.at[pl.ds(s * _SRZ + t * _CHUNK, _CHUNK)])
        pltpu.sync_copy(
            r0.at[pl.ds(0, _SRZ % _CHUNK)],
            acc.at[pl.ds(s * _SRZ + (_SRZ // _CHUNK) * _CHUNK,
                         _SRZ % _CHUNK)])
        plsc.subcore_barrier()

        for sub in range(2):
            lidx = c * _NW + 2 * s + sub
            pltpu.sync_copy(src_idx.at[lidx], src_v)
            pltpu.sync_copy(dst_idx.at[lidx], dst_v)

            def count_body(j, nv):
                lead = dst_v[j, pl.ds(0, _L)]
                return nv + jnp.where(jnp.min(lead) < _TR,
                                      jnp.int32(1), jnp.int32(0))

            nv = lax.fori_loop(0, _CHP, count_body, jnp.int32(0))

            if pipelined:
                @pl.loop(0, _CHP // _NBUF)
                def _(t):
                    @pl.when(t * _NBUF < nv)
                    def _():
                        base = t * _NBUF
                        cps = [pltpu.async_copy(g.at[src_v.at[base + b]],
                                                rows[b], gsems[b])
                               for b in range(_NBUF)]
                        for b in range(_NBUF):
                            cps[b].wait()
                            pltpu.sync_copy(rows[b],
                                            acc.at[dst_v.at[base + b]],
                                            add=True)
            else:
                @pl.loop(0, _CHP)
                def _(j):
                    @pl.when(j < nv)
                    def _():
                        pltpu.async_copy(g.at[src_v.at[j]], r0, gsem).wait()
                        pltpu.sync_copy(r0, acc.at[dst_v.at[j]], add=True)

        plsc.subcore_barrier()
        pltpu.sync_copy(acc.at[pl.ds(s * _SRO, _SRO)],
                        out_hbm.at[pl.ds(c * _H + s * _SRO, _SRO)])

    return k(g_hbm, srcl_hbm, dstl_hbm)


def _mm_body(x_ref, w_ref, o_ref):
    o_ref[...] = jnp.dot(x_ref[...], w_ref[...],
                         preferred_element_type=jnp.float32,
                         precision=lax.Precision.HIGHEST)


def _tc_matmul(x_pad, W):
    d_in, d_out = W.shape
    return pl.pallas_call(
        _mm_body,
        grid=(_N_PAD // _RB,),
        in_specs=[pl.BlockSpec((_RB, d_in), lambda i: (i, 0)),
                  pl.BlockSpec((d_in, d_out), lambda i: (0, 0))],
        out_specs=pl.BlockSpec((_RB, d_out), lambda i: (i, 0)),
        out_shape=jax.ShapeDtypeStruct((_N_PAD, d_out), jnp.float32),
    )(x_pad, W)


def _dis_scale_body(dt_ref, h_ref, dis_ref, g_ref):
    dsum = jnp.sum(dt_ref[...], axis=1, keepdims=True) + 1.0
    dis = lax.rsqrt(dsum)
    dis_ref[...] = dis
    g_ref[...] = h_ref[...] * dis


def _tc_dis_scale(deg_t, h1):
    return pl.pallas_call(
        _dis_scale_body,
        grid=(_N_PAD // _RB,),
        in_specs=[pl.BlockSpec((_RB, _NW), lambda i: (i, 0)),
                  pl.BlockSpec((_RB, _HID), lambda i: (i, 0))],
        out_specs=[pl.BlockSpec((_RB, 1), lambda i: (i, 0)),
                   pl.BlockSpec((_RB, _HID), lambda i: (i, 0))],
        out_shape=[jax.ShapeDtypeStruct((_N_PAD, 1), jnp.float32),
                   jax.ShapeDtypeStruct((_N_PAD, _HID), jnp.float32)],
    )(deg_t, h1)


def _layer1_body(p_ref, g1_ref, dis_ref, b1_ref, w2_ref, g2_ref):
    ssum = p_ref[...] + g1_ref[...]
    z = jnp.maximum(ssum * dis_ref[...] + b1_ref[...], 0.0)
    h2 = jnp.dot(z, w2_ref[...], preferred_element_type=jnp.float32,
                 precision=lax.Precision.HIGHEST)
    g2_ref[...] = h2 * dis_ref[...]


def _tc_layer1_combine(p, g1, dis, b1, W2):
    return pl.pallas_call(
        _layer1_body,
        grid=(_N_PAD // _RB,),
        in_specs=[pl.BlockSpec((_RB, _HID), lambda i: (i, 0)),
                  pl.BlockSpec((_RB, _HID), lambda i: (i, 0)),
                  pl.BlockSpec((_RB, 1), lambda i: (i, 0)),
                  pl.BlockSpec((1, _HID), lambda i: (0, 0)),
                  pl.BlockSpec((_HID, _C), lambda i: (0, 0))],
        out_specs=pl.BlockSpec((_RB, _C), lambda i: (i, 0)),
        out_shape=jax.ShapeDtypeStruct((_N_PAD, _C), jnp.float32),
    )(p, g1, dis, b1, W2)


def _layer2_body(q_ref, g2_ref, dis_ref, b2_ref, o_ref):
    ssum = q_ref[...] + g2_ref[...]
    o_ref[...] = ssum * dis_ref[...] + b2_ref[...]


def _tc_layer2_combine(q, g2, dis, b2):
    return pl.pallas_call(
        _layer2_body,
        grid=(_N_PAD // _RB,),
        in_specs=[pl.BlockSpec((_RB, _C), lambda i: (i, 0)),
                  pl.BlockSpec((_RB, _C), lambda i: (i, 0)),
                  pl.BlockSpec((_RB, 1), lambda i: (i, 0)),
                  pl.BlockSpec((1, _C), lambda i: (0, 0))],
        out_specs=pl.BlockSpec((_RB, _C), lambda i: (i, 0)),
        out_shape=jax.ShapeDtypeStruct((_N_PAD, _C), jnp.float32),
    )(q, g2, dis, b2)


def kernel(x, edge_index, W1, b1, W2, b2):
    src = edge_index[0]
    dst = edge_index[1]
    npad = _E_PAD - _E
    pad_idx = jnp.full((npad,), _N, jnp.int32)
    src_p = jnp.concatenate([src, pad_idx]).reshape(_NW, _CH, _CHUNK)
    dst_p = jnp.concatenate([dst, pad_idx]).reshape(_NW, _CH, _CHUNK)
    x_pad = jnp.zeros((_N_PAD, _D_IN), jnp.float32).at[:_N].set(x)

    deg, s0, d0, s1, d1 = _sc_partition_degree(src_p, dst_p)
    deg_t = deg.T
    srcl = jnp.concatenate([s0, s1]).reshape(2 * _NW, _CHP, _CHUNK)
    dstl = jnp.concatenate([d0, d1]).reshape(2 * _NW, _CHP, _CHUNK)

    h1 = _tc_matmul(x_pad, W1)
    dis, g1 = _tc_dis_scale(deg_t, h1)

    p = _sc_aggregate(g1, srcl, dstl, _HID, pipelined=False)
    g2 = _tc_layer1_combine(p, g1, dis, b1.reshape(1, _HID), W2)

    q = _sc_aggregate(g2, srcl, dstl, _C, linear=True, pipelined=False)
    out = _tc_layer2_combine(q, g2, dis, b2.reshape(1, _C))
    return out[:_N]


# speedup vs baseline: 2.1329x; 1.4426x over previous
import dataclasses
import functools

import jax
import jax.numpy as jnp
from jax import lax
from jax.experimental import pallas as pl
from jax.experimental.pallas import tpu as pltpu
from jax.experimental.pallas import tpu_sc as plsc

_N = 10000
_E = 320000
_D_IN = 128
_HID = 128
_C = 64

_NC = 2
_NS = 16
_NW = _NC * _NS
_L = 16

_N_PAD = 10240
_SR = _N_PAD // _NS
_CHUNK = 128
_CH = 80
_NBUF = 4
_E_PAD = _NW * _CH * _CHUNK
_RB = 1024
_DEG_W = 128
_C_PAD = 128

_CHA = 56
_CHB = 104
_CHT = _CHA + _CHB

_mesh = plsc.VectorSubcoreMesh(core_axis_name="c", subcore_axis_name="s")

_sc_params = pltpu.CompilerParams()
if "needs_layout_passes" in pltpu.CompilerParams.__dataclass_fields__:
    _sc_params = dataclasses.replace(_sc_params, needs_layout_passes=False)
_sc_params = dataclasses.replace(_sc_params, internal_scratch_in_bytes=0)
_sc_linear_params = pltpu.CompilerParams()
if "use_tc_tiling_on_sc" in pltpu.CompilerParams.__dataclass_fields__:
    _sc_linear_params = dataclasses.replace(
        _sc_linear_params, use_tc_tiling_on_sc=False)


def _sc_degree(dst_hbm):
    @functools.partial(
        pl.kernel,
        out_type=jax.ShapeDtypeStruct((_NW, _N_PAD), jnp.float32),
        mesh=_mesh,
        compiler_params=_sc_params,
        scratch_types=[
            pltpu.VMEM((_CH, _CHUNK), jnp.int32),
            pltpu.VMEM((_N_PAD,), jnp.float32),
        ],
    )
    def k(dst_idx, out_hbm, dst_v, hist):
        c = lax.axis_index("c")
        s = lax.axis_index("s")
        w = c * _NS + s

        @pl.loop(0, _N_PAD, step=_L)
        def _(i):
            hist[pl.ds(i, _L)] = jnp.zeros((_L,), jnp.float32)

        pltpu.sync_copy(dst_idx.at[w], dst_v)
        ones = jnp.full((_L,), 1.0, jnp.float32)

        @pl.loop(0, _CH)
        def _(j):
            @pl.loop(0, _CHUNK, step=_L)
            def _(kk):
                idx = dst_v[j, pl.ds(kk, _L)]
                plsc.addupdate_scatter(hist, [idx], ones)

        pltpu.sync_copy(hist, out_hbm.at[w])

    return k(dst_hbm)


def _sc_aggregate(g_hbm, src_hbm, dst_hbm, d, linear=False, pipelined=True):
    @functools.partial(
        pl.kernel,
        out_type=jax.ShapeDtypeStruct((_NC * _N_PAD, d), jnp.float32),
        mesh=_mesh,
        compiler_params=_sc_linear_params if linear else None,
        scratch_types=[
            pltpu.VMEM((_CHB, _CHUNK), jnp.int32),
            pltpu.VMEM((_CHB, _CHUNK), jnp.int32),
            pltpu.VMEM((_CHUNK, d), jnp.float32),
            pltpu.VMEM_SHARED((_N_PAD, d), jnp.float32),
            pltpu.SemaphoreType.DMA,
        ],
    )
    def k(g, src_idx, dst_idx, out_hbm, src_v, dst_v, r0, acc, gsem):
        c = lax.axis_index("c")
        s = lax.axis_index("s")

        @pl.loop(0, _CHUNK)
        def _(i):
            @pl.loop(0, d, step=_L)
            def _(kk):
                r0[i, pl.ds(kk, _L)] = jnp.zeros((_L,), jnp.float32)

        @pl.loop(0, _SR // _CHUNK)
        def _(t):
            pltpu.sync_copy(r0, acc.at[pl.ds(s * _SR + t * _CHUNK, _CHUNK)])

        @pl.when(c == 0)
        def _():
            pltpu.sync_copy(src_idx.at[pl.ds(s * _CHT, _CHA)],
                            src_v.at[pl.ds(0, _CHA)])
            pltpu.sync_copy(dst_idx.at[pl.ds(s * _CHT, _CHA)],
                            dst_v.at[pl.ds(0, _CHA)])

        @pl.when(c == 1)
        def _():
            pltpu.sync_copy(src_idx.at[pl.ds(s * _CHT + _CHA, _CHB)],
                            src_v.at[pl.ds(0, _CHB)])
            pltpu.sync_copy(dst_idx.at[pl.ds(s * _CHT + _CHA, _CHB)],
                            dst_v.at[pl.ds(0, _CHB)])

        plsc.subcore_barrier()

        @pl.when(c == 0)
        def _():
            @pl.loop(0, _CHA)
            def _(j):
                pltpu.async_copy(g.at[src_v.at[j]], r0, gsem).wait()
                pltpu.sync_copy(r0, acc.at[dst_v.at[j]], add=True)

        @pl.when(c == 1)
        def _():
            @pl.loop(0, _CHB)
            def _(j):
                pltpu.async_copy(g.at[src_v.at[j]], r0, gsem).wait()
                pltpu.sync_copy(r0, acc.at[dst_v.at[j]], add=True)

        plsc.subcore_barrier()
        pltpu.sync_copy(acc.at[pl.ds(s * _SR, _SR)],
                        out_hbm.at[pl.ds(c * _N_PAD + s * _SR, _SR)])

    return k(g_hbm, src_hbm, dst_hbm)


def _mm_body(x_ref, w_ref, o_ref):
    o_ref[...] = jnp.dot(x_ref[...], w_ref[...],
                         preferred_element_type=jnp.float32,
                         precision=lax.Precision.HIGHEST)


def _tc_matmul(x_pad, W):
    d_in, d_out = W.shape
    return pl.pallas_call(
        _mm_body,
        grid=(_N_PAD // _RB,),
        in_specs=[pl.BlockSpec((_RB, d_in), lambda i: (i, 0)),
                  pl.BlockSpec((d_in, d_out), lambda i: (0, 0))],
        out_specs=pl.BlockSpec((_RB, d_out), lambda i: (i, 0)),
        out_shape=jax.ShapeDtypeStruct((_N_PAD, d_out), jnp.float32),
    )(x_pad, W)


def _dis_scale_body(dt_ref, h_ref, dis_ref, g_ref):
    dsum = jnp.sum(dt_ref[...], axis=1, keepdims=True) + 1.0
    dis = lax.rsqrt(dsum)
    dis_ref[...] = dis
    g_ref[...] = h_ref[...] * dis


def _tc_dis_scale(deg_t, h1):
    return pl.pallas_call(
        _dis_scale_body,
        grid=(_N_PAD // _RB,),
        in_specs=[pl.BlockSpec((_RB, _NW), lambda i: (i, 0)),
                  pl.BlockSpec((_RB, _HID), lambda i: (i, 0))],
        out_specs=[pl.BlockSpec((_RB, 1), lambda i: (i, 0)),
                   pl.BlockSpec((_RB, _HID), lambda i: (i, 0))],
        out_shape=[jax.ShapeDtypeStruct((_N_PAD, 1), jnp.float32),
                   jax.ShapeDtypeStruct((_N_PAD, _HID), jnp.float32)],
    )(deg_t, h1)


def _layer1_body(p0_ref, p1_ref, g1_ref, dis_ref, b1_ref, w2_ref, g2_ref):
    ssum = p0_ref[0] + p1_ref[0] + g1_ref[...]
    z = jnp.maximum(ssum * dis_ref[...] + b1_ref[...], 0.0)
    h2 = jnp.dot(z, w2_ref[...], preferred_element_type=jnp.float32,
                 precision=lax.Precision.HIGHEST)
    g2_ref[...] = h2 * dis_ref[...]


def _tc_layer1_combine(p, g1, dis, b1, W2):
    return pl.pallas_call(
        _layer1_body,
        grid=(_N_PAD // _RB,),
        in_specs=[pl.BlockSpec((1, _RB, _HID), lambda i: (0, i, 0)),
                  pl.BlockSpec((1, _RB, _HID), lambda i: (1, i, 0)),
                  pl.BlockSpec((_RB, _HID), lambda i: (i, 0)),
                  pl.BlockSpec((_RB, 1), lambda i: (i, 0)),
                  pl.BlockSpec((1, _HID), lambda i: (0, 0)),
                  pl.BlockSpec((_HID, _C), lambda i: (0, 0))],
        out_specs=pl.BlockSpec((_RB, _C), lambda i: (i, 0)),
        out_shape=jax.ShapeDtypeStruct((_N_PAD, _C), jnp.float32),
    )(p, p, g1, dis, b1, W2)


def _layer2_body(q0_ref, q1_ref, g2_ref, dis_ref, b2_ref, o_ref):
    ssum = q0_ref[0] + q1_ref[0] + g2_ref[...]
    o_ref[...] = ssum * dis_ref[...] + b2_ref[...]


def _tc_layer2_combine(q, g2, dis, b2):
    return pl.pallas_call(
        _layer2_body,
        grid=(_N_PAD // _RB,),
        in_specs=[pl.BlockSpec((1, _RB, _C), lambda i: (0, i, 0)),
                  pl.BlockSpec((1, _RB, _C), lambda i: (1, i, 0)),
                  pl.BlockSpec((_RB, _C), lambda i: (i, 0)),
                  pl.BlockSpec((_RB, 1), lambda i: (i, 0)),
                  pl.BlockSpec((1, _C), lambda i: (0, 0))],
        out_specs=pl.BlockSpec((_RB, _C), lambda i: (i, 0)),
        out_shape=jax.ShapeDtypeStruct((_N_PAD, _C), jnp.float32),
    )(q, q, g2, dis, b2)


def kernel(x, edge_index, W1, b1, W2, b2):
    src = edge_index[0]
    dst = edge_index[1]
    npad = _E_PAD - _E
    pad_idx = jnp.full((npad,), _N, jnp.int32)
    src_flat = jnp.concatenate([src, pad_idx])
    dst_flat = jnp.concatenate([dst, pad_idx])
    src_p = src_flat.reshape(_NW, _CH, _CHUNK)
    dst_p = dst_flat.reshape(_NW, _CH, _CHUNK)
    src_a = src_flat.reshape(_NS * _CHT, _CHUNK)
    dst_a = dst_flat.reshape(_NS * _CHT, _CHUNK)
    x_pad = jnp.zeros((_N_PAD, _D_IN), jnp.float32).at[:_N].set(x)

    deg_t = _sc_degree(dst_p).T
    h1 = _tc_matmul(x_pad, W1)
    dis, g1 = _tc_dis_scale(deg_t, h1)

    p = _sc_aggregate(g1, src_a, dst_a, _HID,
                      pipelined=False).reshape(_NC, _N_PAD, _HID)
    g2 = _tc_layer1_combine(p, g1, dis, b1.reshape(1, _HID), W2)

    q = _sc_aggregate(g2, src_a, dst_a, _C, linear=True,
                      pipelined=False).reshape(_NC, _N_PAD, _C)
    out = _tc_layer2_combine(q, g2, dis, b2.reshape(1, _C))
    return out[:_N]


# speedup vs baseline: 2.5985x; 1.2183x over previous
import dataclasses
import functools

import jax
import jax.numpy as jnp
from jax import lax
from jax.experimental import pallas as pl
from jax.experimental.pallas import tpu as pltpu
from jax.experimental.pallas import tpu_sc as plsc

_N = 10000
_E = 320000
_D_IN = 128
_HID = 128
_C = 64

_NC = 2
_NS = 16
_NW = _NC * _NS
_L = 16

_N_PAD = 10240
_SR = _N_PAD // _NS
_CHUNK = 128
_CH = 80
_NBUF = 4
_E_PAD = _NW * _CH * _CHUNK
_RB = 1024
_DEG_W = 128
_C_PAD = 128

_CHA = 112
_CHB = 48
_CHT = _CHA + _CHB
_CHM = max(_CHA, _CHB)

_mesh = plsc.VectorSubcoreMesh(core_axis_name="c", subcore_axis_name="s")

_sc_params = pltpu.CompilerParams()
if "needs_layout_passes" in pltpu.CompilerParams.__dataclass_fields__:
    _sc_params = dataclasses.replace(_sc_params, needs_layout_passes=False)
_sc_params = dataclasses.replace(_sc_params, internal_scratch_in_bytes=0)
_sc_linear_params = pltpu.CompilerParams()
if "use_tc_tiling_on_sc" in pltpu.CompilerParams.__dataclass_fields__:
    _sc_linear_params = dataclasses.replace(
        _sc_linear_params, use_tc_tiling_on_sc=False)


def _sc_degree(dst_hbm):
    @functools.partial(
        pl.kernel,
        out_type=jax.ShapeDtypeStruct((_NW, _N_PAD), jnp.float32),
        mesh=_mesh,
        compiler_params=_sc_params,
        scratch_types=[
            pltpu.VMEM((_CH, _CHUNK), jnp.int32),
            pltpu.VMEM((_N_PAD,), jnp.float32),
        ],
    )
    def k(dst_idx, out_hbm, dst_v, hist):
        c = lax.axis_index("c")
        s = lax.axis_index("s")
        w = c * _NS + s

        @pl.loop(0, _N_PAD, step=_L)
        def _(i):
            hist[pl.ds(i, _L)] = jnp.zeros((_L,), jnp.float32)

        pltpu.sync_copy(dst_idx.at[w], dst_v)
        ones = jnp.full((_L,), 1.0, jnp.float32)

        @pl.loop(0, _CH)
        def _(j):
            @pl.loop(0, _CHUNK, step=_L)
            def _(kk):
                idx = dst_v[j, pl.ds(kk, _L)]
                plsc.addupdate_scatter(hist, [idx], ones)

        pltpu.sync_copy(hist, out_hbm.at[w])

    return k(dst_hbm)


def _sc_aggregate(g_hbm, src_hbm, dst_hbm, d, linear=False, pipelined=True):
    @functools.partial(
        pl.kernel,
        out_type=jax.ShapeDtypeStruct((_NC * _N_PAD, d), jnp.float32),
        mesh=_mesh,
        compiler_params=_sc_linear_params if linear else None,
        scratch_types=[
            pltpu.VMEM((_CHM, _CHUNK), jnp.int32),
            pltpu.VMEM((_CHM, _CHUNK), jnp.int32),
            pltpu.VMEM((_CHUNK, d), jnp.float32),
            pltpu.VMEM_SHARED((_N_PAD, d), jnp.float32),
            pltpu.SemaphoreType.DMA,
        ],
    )
    def k(g, src_idx, dst_idx, out_hbm, src_v, dst_v, r0, acc, gsem):
        c = lax.axis_index("c")
        s = lax.axis_index("s")

        @pl.loop(0, _CHUNK)
        def _(i):
            @pl.loop(0, d, step=_L)
            def _(kk):
                r0[i, pl.ds(kk, _L)] = jnp.zeros((_L,), jnp.float32)

        @pl.loop(0, _SR // _CHUNK)
        def _(t):
            pltpu.sync_copy(r0, acc.at[pl.ds(s * _SR + t * _CHUNK, _CHUNK)])

        @pl.when(c == 0)
        def _():
            pltpu.sync_copy(src_idx.at[pl.ds(s * _CHT, _CHA)],
                            src_v.at[pl.ds(0, _CHA)])
            pltpu.sync_copy(dst_idx.at[pl.ds(s * _CHT, _CHA)],
                            dst_v.at[pl.ds(0, _CHA)])

        @pl.when(c == 1)
        def _():
            pltpu.sync_copy(src_idx.at[pl.ds(s * _CHT + _CHA, _CHB)],
                            src_v.at[pl.ds(0, _CHB)])
            pltpu.sync_copy(dst_idx.at[pl.ds(s * _CHT + _CHA, _CHB)],
                            dst_v.at[pl.ds(0, _CHB)])

        plsc.subcore_barrier()

        @pl.when(c == 0)
        def _():
            @pl.loop(0, _CHA)
            def _(j):
                pltpu.async_copy(g.at[src_v.at[j]], r0, gsem).wait()
                pltpu.sync_copy(r0, acc.at[dst_v.at[j]], add=True)

        @pl.when(c == 1)
        def _():
            @pl.loop(0, _CHB)
            def _(j):
                pltpu.async_copy(g.at[src_v.at[j]], r0, gsem).wait()
                pltpu.sync_copy(r0, acc.at[dst_v.at[j]], add=True)

        plsc.subcore_barrier()
        pltpu.sync_copy(acc.at[pl.ds(s * _SR, _SR)],
                        out_hbm.at[pl.ds(c * _N_PAD + s * _SR, _SR)])

    return k(g_hbm, src_hbm, dst_hbm)


def _mm_body(x_ref, w_ref, o_ref):
    o_ref[...] = jnp.dot(x_ref[...], w_ref[...],
                         preferred_element_type=jnp.float32,
                         precision=lax.Precision.HIGHEST)


def _tc_matmul(x_pad, W):
    d_in, d_out = W.shape
    return pl.pallas_call(
        _mm_body,
        grid=(_N_PAD // _RB,),
        in_specs=[pl.BlockSpec((_RB, d_in), lambda i: (i, 0)),
                  pl.BlockSpec((d_in, d_out), lambda i: (0, 0))],
        out_specs=pl.BlockSpec((_RB, d_out), lambda i: (i, 0)),
        out_shape=jax.ShapeDtypeStruct((_N_PAD, d_out), jnp.float32),
    )(x_pad, W)


def _dis_scale_body(dt_ref, h_ref, dis_ref, g_ref):
    dsum = jnp.sum(dt_ref[...], axis=1, keepdims=True) + 1.0
    dis = lax.rsqrt(dsum)
    dis_ref[...] = dis
    g_ref[...] = h_ref[...] * dis


def _tc_dis_scale(deg_t, h1):
    return pl.pallas_call(
        _dis_scale_body,
        grid=(_N_PAD // _RB,),
        in_specs=[pl.BlockSpec((_RB, _NW), lambda i: (i, 0)),
                  pl.BlockSpec((_RB, _HID), lambda i: (i, 0))],
        out_specs=[pl.BlockSpec((_RB, 1), lambda i: (i, 0)),
                   pl.BlockSpec((_RB, _HID), lambda i: (i, 0))],
        out_shape=[jax.ShapeDtypeStruct((_N_PAD, 1), jnp.float32),
                   jax.ShapeDtypeStruct((_N_PAD, _HID), jnp.float32)],
    )(deg_t, h1)


def _layer1_body(p0_ref, p1_ref, g1_ref, dis_ref, b1_ref, w2_ref, g2_ref):
    ssum = p0_ref[0] + p1_ref[0] + g1_ref[...]
    z = jnp.maximum(ssum * dis_ref[...] + b1_ref[...], 0.0)
    h2 = jnp.dot(z, w2_ref[...], preferred_element_type=jnp.float32,
                 precision=lax.Precision.HIGHEST)
    g2_ref[...] = h2 * dis_ref[...]


def _tc_layer1_combine(p, g1, dis, b1, W2):
    return pl.pallas_call(
        _layer1_body,
        grid=(_N_PAD // _RB,),
        in_specs=[pl.BlockSpec((1, _RB, _HID), lambda i: (0, i, 0)),
                  pl.BlockSpec((1, _RB, _HID), lambda i: (1, i, 0)),
                  pl.BlockSpec((_RB, _HID), lambda i: (i, 0)),
                  pl.BlockSpec((_RB, 1), lambda i: (i, 0)),
                  pl.BlockSpec((1, _HID), lambda i: (0, 0)),
                  pl.BlockSpec((_HID, _C), lambda i: (0, 0))],
        out_specs=pl.BlockSpec((_RB, _C), lambda i: (i, 0)),
        out_shape=jax.ShapeDtypeStruct((_N_PAD, _C), jnp.float32),
    )(p, p, g1, dis, b1, W2)


def _layer2_body(q0_ref, q1_ref, g2_ref, dis_ref, b2_ref, o_ref):
    ssum = q0_ref[0] + q1_ref[0] + g2_ref[...]
    o_ref[...] = ssum * dis_ref[...] + b2_ref[...]


def _tc_layer2_combine(q, g2, dis, b2):
    return pl.pallas_call(
        _layer2_body,
        grid=(_N_PAD // _RB,),
        in_specs=[pl.BlockSpec((1, _RB, _C), lambda i: (0, i, 0)),
                  pl.BlockSpec((1, _RB, _C), lambda i: (1, i, 0)),
                  pl.BlockSpec((_RB, _C), lambda i: (i, 0)),
                  pl.BlockSpec((_RB, 1), lambda i: (i, 0)),
                  pl.BlockSpec((1, _C), lambda i: (0, 0))],
        out_specs=pl.BlockSpec((_RB, _C), lambda i: (i, 0)),
        out_shape=jax.ShapeDtypeStruct((_N_PAD, _C), jnp.float32),
    )(q, q, g2, dis, b2)


def kernel(x, edge_index, W1, b1, W2, b2):
    src = edge_index[0]
    dst = edge_index[1]
    npad = _E_PAD - _E
    pad_idx = jnp.full((npad,), _N, jnp.int32)
    src_flat = jnp.concatenate([src, pad_idx])
    dst_flat = jnp.concatenate([dst, pad_idx])
    src_p = src_flat.reshape(_NW, _CH, _CHUNK)
    dst_p = dst_flat.reshape(_NW, _CH, _CHUNK)
    src_a = src_flat.reshape(_NS * _CHT, _CHUNK)
    dst_a = dst_flat.reshape(_NS * _CHT, _CHUNK)
    x_pad = jnp.zeros((_N_PAD, _D_IN), jnp.float32).at[:_N].set(x)

    deg_t = _sc_degree(dst_p).T
    h1 = _tc_matmul(x_pad, W1)
    dis, g1 = _tc_dis_scale(deg_t, h1)

    p = _sc_aggregate(g1, src_a, dst_a, _HID,
                      pipelined=False).reshape(_NC, _N_PAD, _HID)
    g2 = _tc_layer1_combine(p, g1, dis, b1.reshape(1, _HID), W2)

    q = _sc_aggregate(g2, src_a, dst_a, _C, linear=True,
                      pipelined=False).reshape(_NC, _N_PAD, _C)
    out = _tc_layer2_combine(q, g2, dis, b2.reshape(1, _C))
    return out[:_N]
